# baseline (speedup 1.0000x reference)
import functools

import jax
import jax.numpy as jnp
from jax import lax
from jax.experimental import pallas as pl
from jax.experimental.pallas import tpu as pltpu
from jax.experimental.pallas import tpu_sc as plsc

EMBED = 64
HALVES = ((0, 104), (104, 96))
IDX_BLK = 16


def _make_kernel(batch: int, seq: int, num_cores: int, num_subcores: int):
    nw = num_cores * num_subcores
    rows_per_w = batch // nw
    n_blocks = rows_per_w // IDX_BLK
    n_pairs = IDX_BLK // 2

    mesh = plsc.VectorSubcoreMesh(core_axis_name="c", subcore_axis_name="s")

    @functools.partial(
        pl.kernel,
        mesh=mesh,
        compiler_params=pltpu.CompilerParams(use_tc_tiling_on_sc=False),
        out_type=jax.ShapeDtypeStruct((batch, seq, EMBED), jnp.float32),
        scratch_types=[
            pltpu.VMEM((IDX_BLK, seq), jnp.int32),
            pltpu.VMEM((IDX_BLK, seq), jnp.int32),
            pltpu.VMEM((IDX_BLK, seq), jnp.int32),
            pltpu.VMEM((seq, EMBED), jnp.float32),
            pltpu.VMEM((seq, EMBED), jnp.float32),
            pltpu.VMEM((seq, EMBED), jnp.float32),
            pltpu.VMEM((seq, EMBED), jnp.float32),
            pltpu.VMEM((seq, EMBED), jnp.float32),
            pltpu.VMEM((seq, EMBED), jnp.float32),
            pltpu.VMEM((seq, EMBED), jnp.float32),
            pltpu.VMEM((seq, EMBED), jnp.float32),
            pltpu.SemaphoreType.DMA,
            pltpu.SemaphoreType.DMA,
            pltpu.SemaphoreType.DMA,
            pltpu.SemaphoreType.DMA,
        ],
    )
    def k(seq_hbm, pos_hbm, pap_hbm, tok_tab, pos_tab, pap_tab, out_hbm,
          idx_t, idx_p, idx_q, tok0, tok1, pos0, pos1, pap0, pap1,
          sum0, sum1, gsem0, gsem1, wsem0, wsem1):
        wid = lax.axis_index("s") * num_cores + lax.axis_index("c")
        row0 = wid * rows_per_w
        tok_b, pos_b, pap_b = (tok0, tok1), (pos0, pos1), (pap0, pap1)
        sum_b = (sum0, sum1)
        gsem = (gsem0, gsem1)
        wsem = (wsem0, wsem1)

        def fire_gathers(lr, b):
            for start, size in HALVES:
                sl = pl.ds(start, size)
                pltpu.async_copy(tok_tab.at[idx_t.at[lr, sl]], tok_b[b].at[sl], gsem[b])
                pltpu.async_copy(pos_tab.at[idx_p.at[lr, sl]], pos_b[b].at[sl], gsem[b])
                pltpu.async_copy(pap_tab.at[idx_q.at[lr, sl]], pap_b[b].at[sl], gsem[b])

        def wait_gathers(b):
            dummy = out_hbm.at[0]
            pltpu.make_async_copy(dummy, tok_b[b], gsem[b]).wait()
            pltpu.make_async_copy(dummy, pos_b[b], gsem[b]).wait()
            pltpu.make_async_copy(dummy, pap_b[b], gsem[b]).wait()

        def fire_write(gr, b):
            pltpu.async_copy(sum_b[b], out_hbm.at[gr], wsem[b])

        def wait_write(b):
            pltpu.make_async_copy(sum_b[b], out_hbm.at[0], wsem[b]).wait()

        def compute(b):
            tok, pos, pap, acc = tok_b[b], pos_b[b], pap_b[b], sum_b[b]

            def add_body(i, carry):
                for j in range(EMBED // 16):
                    sl = pl.ds(j * 16, 16)
                    acc[i, sl] = tok[i, sl] + pos[i, sl] + pap[i, sl]
                return carry

            lax.fori_loop(0, seq, add_body, None)

        for blk in range(n_blocks):
            blk_row0 = row0 + blk * IDX_BLK
            pltpu.sync_copy(seq_hbm.at[pl.ds(blk_row0, IDX_BLK)], idx_t)
            pltpu.sync_copy(pos_hbm.at[pl.ds(blk_row0, IDX_BLK)], idx_p)
            pltpu.sync_copy(pap_hbm.at[pl.ds(blk_row0, IDX_BLK)], idx_q)

            for b in (0, 1):
                fire_gathers(b, b)

            def pair_body(p, carry):
                for b in (0, 1):
                    lr = 2 * p + b
                    wait_gathers(b)
                    wait_write(b)
                    compute(b)
                    fire_write(blk_row0 + lr, b)

                    @pl.when(lr + 2 < IDX_BLK)
                    def _():
                        fire_gathers(lr + 2, b)
                return carry

            if blk == 0:
                for b in (0, 1):
                    wait_gathers(b)
                    compute(b)
                    fire_write(blk_row0 + b, b)
                    fire_gathers(2 + b, b)
                lax.fori_loop(1, n_pairs, pair_body, None)
            else:
                lax.fori_loop(0, n_pairs, pair_body, None)

        for b in (0, 1):
            wait_write(b)

    return k


def kernel(sequence, position_ids, paper_ids, token_table, position_table, paper_table):
    batch, seq = sequence.shape
    info = plsc.get_sparse_core_info()
    num_cores, num_subcores = info.num_cores, info.num_subcores
    assert seq == sum(size for _, size in HALVES)
    assert batch % (num_cores * num_subcores * IDX_BLK) == 0

    k = _make_kernel(batch, seq, num_cores, num_subcores)
    return k(sequence.astype(jnp.int32), position_ids.astype(jnp.int32),
             paper_ids.astype(jnp.int32), token_table, position_table, paper_table)
